# baseline (speedup 1.0000x reference)
import functools

import jax
import jax.numpy as jnp
import numpy as np
from jax.experimental import pallas as pl
from jax.experimental.pallas import tpu as pltpu
from jax.experimental.pallas import tpu_sc as plsc

_D = 64
_SEQ = 200
_SCALE = 8.0

_CHUNK = 128
_TC_BATCH = 128


def _pe_table() -> np.ndarray:
    half = _D / 2
    positions = np.arange(_SEQ)[:, np.newaxis]
    depths = np.arange(half)[np.newaxis, :] / half
    angle_rads = positions * (1.0 / 10000**depths)
    return np.concatenate(
        [np.sin(angle_rads), np.cos(angle_rads)], axis=-1
    ).astype(np.float32)


def _sc_gather(table, idx):
    n = idx.shape[1]
    nw = 32
    per_w = n // nw
    nchunks = per_w // _CHUNK
    mesh = plsc.VectorSubcoreMesh(core_axis_name="core", subcore_axis_name="subcore")

    @functools.partial(
        pl.kernel,
        out_type=jax.ShapeDtypeStruct((n, _D), table.dtype),
        mesh=mesh,
        scratch_types=[
            pltpu.VMEM((1, per_w), jnp.int32),
            pltpu.VMEM((_CHUNK, _D), jnp.float32),
            pltpu.VMEM((_CHUNK, _D), jnp.float32),
            pltpu.SemaphoreType.DMA,
            pltpu.SemaphoreType.DMA,
            pltpu.SemaphoreType.DMA,
            pltpu.SemaphoreType.DMA,
        ],
        compiler_params=pltpu.CompilerParams(use_tc_tiling_on_sc=False),
    )
    def k(table_hbm, i_hbm, o_hbm, idx_v, bufa, bufb, sga, sgb, swa, swb):
        wid = jax.lax.axis_index("core") * 16 + jax.lax.axis_index("subcore")
        base = wid * per_w
        pltpu.sync_copy(i_hbm.at[:, pl.ds(base, per_w)], idx_v)

        def g_start(c, buf, sem):
            pltpu.async_copy(
                table_hbm.at[idx_v.at[0, pl.ds(c * _CHUNK, _CHUNK)]], buf, sem
            )

        def g_wait(buf, sem):
            pltpu.make_async_copy(table_hbm.at[idx_v.at[0, pl.ds(0, _CHUNK)]], buf, sem).wait()

        def w_start(c, buf, sem):
            pltpu.async_copy(buf, o_hbm.at[pl.ds(base + c * _CHUNK, _CHUNK)], sem)

        def w_wait(buf, sem):
            pltpu.make_async_copy(buf, o_hbm.at[pl.ds(base, _CHUNK)], sem).wait()

        g_start(0, bufa, sga)
        g_start(1, bufb, sgb)

        @pl.loop(0, nchunks, step=2)
        def _(c):
            g_wait(bufa, sga)
            w_start(c, bufa, swa)
            g_wait(bufb, sgb)
            w_start(c + 1, bufb, swb)

            @pl.when(c + 2 < nchunks)
            def _():
                w_wait(bufa, swa)
                g_start(c + 2, bufa, sga)

            @pl.when(c + 3 < nchunks)
            def _():
                w_wait(bufb, swb)
                g_start(c + 3, bufb, sgb)

        w_wait(bufa, swa)
        w_wait(bufb, swb)

    return k(table, idx)


_L_CHUNK = 8


def _fixup_body(g_ref, pe_ref, o_ref, *, b):
    gb = g_ref[...].reshape(_L_CHUNK, b, _D)
    t = jnp.transpose(gb, (0, 2, 1))
    pe = pe_ref[...][:, :, None]
    o_ref[...] = t * _SCALE + pe


def _tc_fixup(g, pe, b):
    return pl.pallas_call(
        functools.partial(_fixup_body, b=b),
        grid=(_SEQ // _L_CHUNK,),
        in_specs=[
            pl.BlockSpec((_L_CHUNK * b, _D), lambda i: (i, 0)),
            pl.BlockSpec((_L_CHUNK, _D), lambda i: (i, 0)),
        ],
        out_specs=pl.BlockSpec((_L_CHUNK, _D, b), lambda i: (i, 0, 0)),
        out_shape=jax.ShapeDtypeStruct((_SEQ, _D, b), jnp.float32),
    )(g, pe)


def kernel(x, table):
    b, l = x.shape
    idx = x.T.reshape(1, b * l).astype(jnp.int32)
    g = _sc_gather(table, idx)
    pe = jnp.asarray(_pe_table())
    out3 = _tc_fixup(g, pe, b)
    return out3.transpose(2, 0, 1)


# speedup vs baseline: 1.4582x; 1.4582x over previous
import functools

import jax
import jax.numpy as jnp
import numpy as np
from jax.experimental import pallas as pl
from jax.experimental.pallas import tpu as pltpu
from jax.experimental.pallas import tpu_sc as plsc

_D = 64
_SEQ = 200
_SCALE = 8.0

_CHUNK = 128
_PAIR_D = 128


def _pe_table() -> np.ndarray:
    half = _D / 2
    positions = np.arange(_SEQ)[:, np.newaxis]
    depths = np.arange(half)[np.newaxis, :] / half
    angle_rads = positions * (1.0 / 10000**depths)
    return np.concatenate(
        [np.sin(angle_rads), np.cos(angle_rads)], axis=-1
    ).astype(np.float32)


def _sc_gather(table2, idx):
    n = idx.shape[1]
    nw = 32
    per_w = n // nw
    nchunks = per_w // _CHUNK
    mesh = plsc.VectorSubcoreMesh(core_axis_name="core", subcore_axis_name="subcore")

    @functools.partial(
        pl.kernel,
        out_type=jax.ShapeDtypeStruct((n, _PAIR_D), table2.dtype),
        mesh=mesh,
        scratch_types=[
            pltpu.VMEM((1, per_w), jnp.int32),
            pltpu.VMEM((_CHUNK, _PAIR_D), jnp.float32),
            pltpu.VMEM((_CHUNK, _PAIR_D), jnp.float32),
            pltpu.SemaphoreType.DMA,
            pltpu.SemaphoreType.DMA,
            pltpu.SemaphoreType.DMA,
            pltpu.SemaphoreType.DMA,
        ],
        compiler_params=pltpu.CompilerParams(use_tc_tiling_on_sc=False),
    )
    def k(table_hbm, i_hbm, o_hbm, idx_v, bufa, bufb, sga, sgb, swa, swb):
        wid = jax.lax.axis_index("core") * 16 + jax.lax.axis_index("subcore")
        base = wid * per_w
        pltpu.sync_copy(i_hbm.at[:, pl.ds(base, per_w)], idx_v)

        def g_start(c, buf, sem):
            pltpu.async_copy(
                table_hbm.at[idx_v.at[0, pl.ds(c * _CHUNK, _CHUNK)]], buf, sem
            )

        def g_wait(buf, sem):
            pltpu.make_async_copy(table_hbm.at[idx_v.at[0, pl.ds(0, _CHUNK)]], buf, sem).wait()

        def w_start(c, buf, sem):
            pltpu.async_copy(buf, o_hbm.at[pl.ds(base + c * _CHUNK, _CHUNK)], sem)

        def w_wait(buf, sem):
            pltpu.make_async_copy(buf, o_hbm.at[pl.ds(base, _CHUNK)], sem).wait()

        g_start(0, bufa, sga)
        g_start(1, bufb, sgb)

        @pl.loop(0, nchunks, step=2)
        def _(c):
            g_wait(bufa, sga)
            w_start(c, bufa, swa)
            g_wait(bufb, sgb)
            w_start(c + 1, bufb, swb)

            @pl.when(c + 2 < nchunks)
            def _():
                w_wait(bufa, swa)
                g_start(c + 2, bufa, sga)

            @pl.when(c + 3 < nchunks)
            def _():
                w_wait(bufb, swb)
                g_start(c + 3, bufb, sgb)

        w_wait(bufa, swa)
        w_wait(bufb, swb)

    return k(table2, idx)


_FMT_V = 8192


def _fmt_body(tT_ref, o_ref):
    blk = tT_ref[...]
    tr = jnp.transpose(blk)
    r3 = tr.reshape(_FMT_V // 2, 2, _D)
    o_ref[:, :_D] = r3[:, 0, :]
    o_ref[:, _D:] = r3[:, 1, :]


def _tc_format(tT):
    v = tT.shape[1]
    return pl.pallas_call(
        _fmt_body,
        grid=(pl.cdiv(v, _FMT_V),),
        in_specs=[pl.BlockSpec((_D, _FMT_V), lambda i: (0, i))],
        out_specs=pl.BlockSpec((_FMT_V // 2, 2 * _D), lambda i: (i, 0)),
        out_shape=jax.ShapeDtypeStruct((v // 2, 2 * _D), jnp.float32),
    )(tT)


_L_CHUNK = 8


def _fixup_body(g_ref, par_ref, pe_ref, o_ref, *, b):
    gb = g_ref[...].reshape(_L_CHUNK, b, _PAIR_D)
    tr = jnp.transpose(gb, (0, 2, 1))
    lo = tr[:, :_D, :]
    hi = tr[:, _D:, :]
    par = (par_ref[...] != 0)[:, None, :]
    pe = pe_ref[...][:, :, None]
    o_ref[...] = jnp.where(par, hi, lo) * _SCALE + pe


def _tc_fixup(g, par, pe, b):
    return pl.pallas_call(
        functools.partial(_fixup_body, b=b),
        grid=(_SEQ // _L_CHUNK,),
        in_specs=[
            pl.BlockSpec((_L_CHUNK * b, _PAIR_D), lambda i: (i, 0)),
            pl.BlockSpec((_L_CHUNK, b), lambda i: (i, 0)),
            pl.BlockSpec((_L_CHUNK, _D), lambda i: (i, 0)),
        ],
        out_specs=pl.BlockSpec((_L_CHUNK, _D, b), lambda i: (i, 0, 0)),
        out_shape=jax.ShapeDtypeStruct((_SEQ, _D, b), jnp.float32),
    )(g, par, pe)


def kernel(x, table):
    b, l = x.shape
    xt = x.T.astype(jnp.int32)
    idx2 = (xt >> 1).reshape(1, b * l)
    par = xt & 1
    table2 = _tc_format(table.T)
    g = _sc_gather(table2, idx2)
    pe = jnp.asarray(_pe_table())
    out3 = _tc_fixup(g, par, pe, b)
    return out3.transpose(2, 0, 1)


# speedup vs baseline: 1.5050x; 1.0321x over previous
import functools

import jax
import jax.numpy as jnp
import numpy as np
from jax.experimental import pallas as pl
from jax.experimental.pallas import tpu as pltpu
from jax.experimental.pallas import tpu_sc as plsc

_D = 64
_SEQ = 200
_SCALE = 8.0

_CHUNK = 128
_PAIR_D = 128


def _pe_table() -> np.ndarray:
    half = _D / 2
    positions = np.arange(_SEQ)[:, np.newaxis]
    depths = np.arange(half)[np.newaxis, :] / half
    angle_rads = positions * (1.0 / 10000**depths)
    return np.concatenate(
        [np.sin(angle_rads), np.cos(angle_rads)], axis=-1
    ).astype(np.float32)


def _sc_gather(table2, idx):
    n = idx.shape[1]
    nw = 32
    per_w = n // nw
    nchunks = per_w // _CHUNK
    mesh = plsc.VectorSubcoreMesh(core_axis_name="core", subcore_axis_name="subcore")

    @functools.partial(
        pl.kernel,
        out_type=jax.ShapeDtypeStruct((n, _PAIR_D), table2.dtype),
        mesh=mesh,
        scratch_types=[
            pltpu.VMEM((1, per_w), jnp.int32),
            pltpu.VMEM((_CHUNK, _PAIR_D), jnp.float32),
            pltpu.VMEM((_CHUNK, _PAIR_D), jnp.float32),
            pltpu.SemaphoreType.DMA,
            pltpu.SemaphoreType.DMA,
            pltpu.SemaphoreType.DMA,
            pltpu.SemaphoreType.DMA,
        ],
        compiler_params=pltpu.CompilerParams(use_tc_tiling_on_sc=False),
    )
    def k(table_hbm, i_hbm, o_hbm, idx_v, bufa, bufb, sga, sgb, swa, swb):
        wid = jax.lax.axis_index("core") * 16 + jax.lax.axis_index("subcore")
        base = wid * per_w
        pltpu.sync_copy(i_hbm.at[:, pl.ds(base, per_w)], idx_v)

        def g_start(c, buf, sem):
            pltpu.async_copy(
                table_hbm.at[idx_v.at[0, pl.ds(c * _CHUNK, _CHUNK)]], buf, sem
            )

        def g_wait(buf, sem):
            pltpu.make_async_copy(table_hbm.at[idx_v.at[0, pl.ds(0, _CHUNK)]], buf, sem).wait()

        def w_start(c, buf, sem):
            pltpu.async_copy(buf, o_hbm.at[pl.ds(base + c * _CHUNK, _CHUNK)], sem)

        def w_wait(buf, sem):
            pltpu.make_async_copy(buf, o_hbm.at[pl.ds(base, _CHUNK)], sem).wait()

        g_start(0, bufa, sga)
        g_start(1, bufb, sgb)

        @pl.loop(0, nchunks, step=2)
        def _(c):
            g_wait(bufa, sga)
            w_start(c, bufa, swa)
            g_wait(bufb, sgb)
            w_start(c + 1, bufb, swb)

            @pl.when(c + 2 < nchunks)
            def _():
                w_wait(bufa, swa)
                g_start(c + 2, bufa, sga)

            @pl.when(c + 3 < nchunks)
            def _():
                w_wait(bufb, swb)
                g_start(c + 3, bufb, sgb)

        w_wait(bufa, swa)
        w_wait(bufb, swb)

    return k(table2, idx)


_FMT_V = 16384


def _fmt_body(tT_ref, o_ref):
    blk = tT_ref[...]
    tr = jnp.transpose(blk)
    r3 = tr.reshape(_FMT_V // 2, 2, _D)
    o_ref[:, :_D] = r3[:, 0, :]
    o_ref[:, _D:] = r3[:, 1, :]


def _tc_format(tT):
    v = tT.shape[1]
    return pl.pallas_call(
        _fmt_body,
        grid=(pl.cdiv(v, _FMT_V),),
        in_specs=[pl.BlockSpec((_D, _FMT_V), lambda i: (0, i))],
        out_specs=pl.BlockSpec((_FMT_V // 2, 2 * _D), lambda i: (i, 0)),
        out_shape=jax.ShapeDtypeStruct((v // 2, 2 * _D), jnp.float32),
    )(tT)


_L_CHUNK = 8


def _fixup_body(g_ref, par_ref, pe_ref, o_ref, *, b):
    gb = g_ref[...].reshape(_L_CHUNK, b, _PAIR_D)
    tr = jnp.transpose(gb, (0, 2, 1))
    lo = tr[:, :_D, :]
    hi = tr[:, _D:, :]
    par = (par_ref[...] != 0)[:, None, :]
    pe = pe_ref[...][:, :, None]
    o_ref[...] = jnp.where(par, hi, lo) * _SCALE + pe


def _tc_fixup(g, par, pe, b):
    return pl.pallas_call(
        functools.partial(_fixup_body, b=b),
        grid=(_SEQ // _L_CHUNK,),
        in_specs=[
            pl.BlockSpec((_L_CHUNK * b, _PAIR_D), lambda i: (i, 0)),
            pl.BlockSpec((_L_CHUNK, b), lambda i: (i, 0)),
            pl.BlockSpec((_L_CHUNK, _D), lambda i: (i, 0)),
        ],
        out_specs=pl.BlockSpec((_L_CHUNK, _D, b), lambda i: (i, 0, 0)),
        out_shape=jax.ShapeDtypeStruct((_SEQ, _D, b), jnp.float32),
    )(g, par, pe)


def kernel(x, table):
    b, l = x.shape
    xt = x.T.astype(jnp.int32)
    idx2 = (xt >> 1).reshape(1, b * l)
    par = xt & 1
    table2 = _tc_format(table.T)
    g = _sc_gather(table2, idx2)
    pe = jnp.asarray(_pe_table())
    out3 = _tc_fixup(g, par, pe, b)
    return out3.transpose(2, 0, 1)
